# baseline (speedup 1.0000x reference)
import functools

import jax
import jax.numpy as jnp
from jax import lax
from jax.experimental import pallas as pl
from jax.experimental.pallas import tpu as pltpu
from jax.experimental.pallas import tpu_sc as plsc

NU = 50000
NI = 50000
NN = NU + NI
D = 32
HALF = NN // 2
NC, NS = 2, 16
NW = NC * NS

SB = 1024
GB = 128
NGB = SB // GB
NBUF = 4
ACC_ROWS = 51200
ZSTRIPE = ACC_ROWS // NS
WB = HALF // NS


def _spmm_kernel(nsb):
  ept = nsb * SB
  mesh = plsc.VectorSubcoreMesh(core_axis_name="c", subcore_axis_name="s")

  @functools.partial(
      pl.kernel,
      mesh=mesh,
      out_type=jax.ShapeDtypeStruct((NN, D), jnp.float32),
      compiler_params=pltpu.CompilerParams(
          use_tc_tiling_on_sc=False, needs_layout_passes=False),
      scratch_types=[
          pltpu.VMEM((2, SB), jnp.int32),
          pltpu.VMEM((2, SB), jnp.int32),
          pltpu.VMEM((2, SB), jnp.float32),
          pltpu.VMEM((SB + GB,), jnp.int32),
          pltpu.VMEM((SB + GB,), jnp.int32),
          pltpu.VMEM((SB + GB,), jnp.float32),
          pltpu.VMEM((2, GB), jnp.int32),
          pltpu.VMEM((2 * GB, D), jnp.float32),
          pltpu.VMEM_SHARED((ACC_ROWS, D), jnp.float32),
          pltpu.SemaphoreType.DMA,
          pltpu.SemaphoreType.DMA,
          pltpu.SemaphoreType.DMA,
          pltpu.SemaphoreType.DMA,
          pltpu.SemaphoreType.DMA,
      ],
  )
  def body(emb, srcs, dsts, vals, out, src_v, dst_v, val_v, csrc, cdl, cval,
           dloc_v, rows_v, acc, ssem, g0, g1, c0, c1):
    gsems = (g0, g1)
    csems = (c0, c1)
    c = lax.axis_index("c")
    s = lax.axis_index("s")
    zero16 = jnp.zeros((16,), jnp.float32)

    def zrow(i, _):
      rows_v[i, pl.ds(0, 16)] = zero16
      rows_v[i, pl.ds(16, 16)] = zero16
      return 0

    lax.fori_loop(0, GB, zrow, 0)

    def zacc(b, _):
      pltpu.sync_copy(rows_v.at[pl.ds(0, GB)],
                      acc.at[pl.ds(s * ZSTRIPE + b * GB, GB)])
      return 0

    lax.fori_loop(0, ZSTRIPE // GB, zacc, 0)
    plsc.subcore_barrier()

    cbase = c * HALF
    lane = lax.iota(jnp.int32, 16)
    zero16i = jnp.zeros((16,), jnp.int32)
    dummy16 = zero16i + HALF

    def stage(b, buf):
      base = s * ept + b * SB
      pltpu.async_copy(srcs.at[pl.ds(base, SB)], src_v.at[buf], ssem)
      pltpu.async_copy(dsts.at[pl.ds(base, SB)], dst_v.at[buf], ssem)
      pltpu.async_copy(vals.at[pl.ds(base, SB)], val_v.at[buf], ssem)

    stage(0, 0)

    def process_block(k, sub):
      boff = sub * GB
      rows_sl = rows_v.at[pl.ds(boff, GB)]
      coff = k * GB
      pltpu.make_async_copy(emb.at[pl.ds(0, GB)], rows_sl, gsems[sub]).wait()

      def scale(q, _):
        eb = q * 16
        vv = cval[pl.ds(coff + eb, 16)]
        dloc_v[sub, pl.ds(q * 16, 16)] = cdl[pl.ds(coff + eb, 16)]
        for u in range(16):
          r = boff + eb + u
          rows_v[r, pl.ds(0, 16)] = rows_v[r, pl.ds(0, 16)] * vv[u]
          rows_v[r, pl.ds(16, 16)] = rows_v[r, pl.ds(16, 16)] * vv[u]
        return 0

      lax.fori_loop(0, GB // 16, scale, 0)

      pltpu.async_copy(rows_sl, acc.at[dloc_v.at[sub]], csems[sub], add=True)

      @pl.when(k >= 1)
      def _():
        pltpu.make_async_copy(emb.at[pl.ds(0, GB)],
                              acc.at[pl.ds(0, GB)], csems[1 - sub]).wait()

    def super_body(b, _):
      buf = lax.rem(b, 2)
      pltpu.make_async_copy(srcs.at[pl.ds(0, SB)], src_v.at[buf], ssem).wait()
      pltpu.make_async_copy(dsts.at[pl.ds(0, SB)], dst_v.at[buf], ssem).wait()
      pltpu.make_async_copy(vals.at[pl.ds(0, SB)], val_v.at[buf], ssem).wait()

      @pl.when(b < nsb - 1)
      def _():
        stage(b + 1, 1 - buf)

      def compact(g, woff):
        sl = pl.ds(g * 16, 16)
        sv = src_v[buf, sl]
        dv = dst_v[buf, sl]
        vv = val_v[buf, sl]
        dl = dv - cbase
        ok = (dl >= 0) & (dl < HALF)
        xi = jnp.where(ok, 1, 0)
        cum = plsc.cumsum(xi)
        idx = (woff + cum) - xi
        plsc.store_scatter(csrc, [idx], sv, mask=ok)
        plsc.store_scatter(cdl, [idx], dl, mask=ok)
        plsc.store_scatter(cval, [idx], vv, mask=ok)
        return woff + cum[15]

      woff = lax.fori_loop(0, SB // 16, compact, jnp.int32(0))

      for g in range(GB // 16):
        idx = woff + lane + g * 16
        plsc.store_scatter(csrc, [idx], zero16i)
        plsc.store_scatter(cdl, [idx], dummy16)
        plsc.store_scatter(cval, [idx], zero16)

      nblk = (woff + (GB - 1)) >> 7

      @pl.when(nblk > 0)
      def _():
        pltpu.async_copy(emb.at[csrc.at[pl.ds(0, GB)]],
                         rows_v.at[pl.ds(0, GB)], gsems[0])

      for k in range(NGB):
        sub = k % 2

        @pl.when(k < nblk)
        def _(k=k, sub=sub):
          process_block(k, sub)

          @pl.when(k + 1 < nblk)
          def _():
            pltpu.async_copy(
                emb.at[csrc.at[pl.ds((k + 1) * GB, GB)]],
                rows_v.at[pl.ds((1 - sub) * GB, GB)], gsems[1 - sub])

      @pl.when(nblk > 0)
      def _():
        par = lax.rem(nblk - 1, 2)

        @pl.when(par == 0)
        def _():
          pltpu.make_async_copy(emb.at[pl.ds(0, GB)],
                                acc.at[pl.ds(0, GB)], csems[0]).wait()

        @pl.when(par == 1)
        def _():
          pltpu.make_async_copy(emb.at[pl.ds(0, GB)],
                                acc.at[pl.ds(0, GB)], csems[1]).wait()

      return 0

    lax.fori_loop(0, nsb, super_body, 0)

    plsc.subcore_barrier()
    pltpu.sync_copy(acc.at[pl.ds(s * 3128, 3080)],
                    out.at[pl.ds(cbase + s * 3128, 3080)])

    @pl.when(s < NS - 1)
    def _():
      pltpu.sync_copy(acc.at[pl.ds(s * 3128 + 3080, 48)],
                      out.at[pl.ds(cbase + s * 3128 + 3080, 48)])

  return body


def _final_kernel(batch):
  pb = batch // NW
  mesh = plsc.VectorSubcoreMesh(core_axis_name="c", subcore_axis_name="s")

  @functools.partial(
      pl.kernel,
      mesh=mesh,
      out_type=jax.ShapeDtypeStruct((batch,), jnp.float32),
      compiler_params=pltpu.CompilerParams(
          use_tc_tiling_on_sc=False, needs_layout_passes=False),
      scratch_types=[
          pltpu.VMEM((pb,), jnp.int32),
          pltpu.VMEM((pb,), jnp.int32),
          pltpu.VMEM((pb,), jnp.int32),
          pltpu.VMEM((pb, D), jnp.float32),
          pltpu.VMEM((pb, D), jnp.float32),
          pltpu.VMEM((pb, D), jnp.float32),
          pltpu.VMEM((pb,), jnp.float32),
          pltpu.SemaphoreType.DMA,
      ],
  )
  def body(ut, it, e1, e2, e3, users, items, out,
           uidx_v, iidx_v, iidx2_v, au, ai, tmp, gam, sem):
    c = lax.axis_index("c")
    s = lax.axis_index("s")
    wid = s * NC + c
    base = wid * pb
    pltpu.sync_copy(users.at[pl.ds(base, pb)], uidx_v)
    pltpu.sync_copy(items.at[pl.ds(base, pb)], iidx_v)
    for i in range(pb // 16):
      iidx2_v[pl.ds(i * 16, 16)] = iidx_v[pl.ds(i * 16, 16)] + NU

    pltpu.async_copy(ut.at[uidx_v], au, sem).wait()
    pltpu.async_copy(it.at[iidx_v], ai, sem).wait()

    for tab in (e1, e2, e3):
      pltpu.async_copy(tab.at[uidx_v], tmp, sem).wait()

      def addu(r, _):
        au[r, pl.ds(0, 16)] = au[r, pl.ds(0, 16)] + tmp[r, pl.ds(0, 16)]
        au[r, pl.ds(16, 16)] = au[r, pl.ds(16, 16)] + tmp[r, pl.ds(16, 16)]
        return 0

      lax.fori_loop(0, pb, addu, 0)
      pltpu.async_copy(tab.at[iidx2_v], tmp, sem).wait()

      def addi(r, _):
        ai[r, pl.ds(0, 16)] = ai[r, pl.ds(0, 16)] + tmp[r, pl.ds(0, 16)]
        ai[r, pl.ds(16, 16)] = ai[r, pl.ds(16, 16)] + tmp[r, pl.ds(16, 16)]
        return 0

      lax.fori_loop(0, pb, addi, 0)

    lane = lax.iota(jnp.int32, 16)
    quarter = jnp.float32(0.25)
    one = jnp.float32(1.0)

    def outer(o, _):
      def inner(k, carry):
        numvec, denvec = carry
        r = o * 16 + k
        u0 = au[r, pl.ds(0, 16)] * quarter
        u1 = au[r, pl.ds(16, 16)] * quarter
        s0 = one / (one + jnp.exp(-u0))
        s1 = one / (one + jnp.exp(-u1))
        i0 = ai[r, pl.ds(0, 16)] * quarter
        i1 = ai[r, pl.ds(16, 16)] * quarter
        x0 = jnp.exp(i0)
        x1 = jnp.exp(i1)
        den = jnp.sum(x0) + jnp.sum(x1)
        num = jnp.sum(s0 * x0) + jnp.sum(s1 * x1)
        hit = lane == k
        return (jnp.where(hit, num, numvec), jnp.where(hit, den, denvec))

      z16 = jnp.zeros((16,), jnp.float32)
      numvec, denvec = lax.fori_loop(0, 16, inner, (z16, z16 + one))
      gam[pl.ds(o * 16, 16)] = numvec / denvec
      return 0

    lax.fori_loop(0, pb // 16, outer, 0)
    pltpu.sync_copy(gam, out.at[pl.ds(base, pb)])

  return body


def kernel(users, items, user_table, item_table, edge_index, edge_vals):
  all0 = jnp.concatenate([user_table, item_table], axis=0)
  ne = edge_vals.shape[0]
  nsb = -(-ne // (NS * SB))
  pad = nsb * NS * SB - ne
  src = jnp.concatenate([edge_index[0], jnp.zeros((pad,), jnp.int32)])
  dst = jnp.concatenate([edge_index[1], jnp.zeros((pad,), jnp.int32)])
  val = jnp.concatenate([edge_vals, jnp.zeros((pad,), jnp.float32)])

  spmm = _spmm_kernel(nsb)
  e1 = spmm(all0, src, dst, val)
  e2 = spmm(e1, src, dst, val)
  e3 = spmm(e2, src, dst, val)
  fin = _final_kernel(users.shape[0])
  return fin(user_table, item_table, e1, e2, e3, users, items)


# speedup vs baseline: 2.4451x; 2.4451x over previous
import functools

import jax
import jax.numpy as jnp
from jax import lax
from jax.experimental import pallas as pl
from jax.experimental.pallas import tpu as pltpu
from jax.experimental.pallas import tpu_sc as plsc

NU = 50000
NI = 50000
NN = NU + NI
D = 32
HALF = NN // 2
NC, NS = 2, 16
NW = NC * NS

SB = 1024
GB = 128
NGB = SB // GB
RING = 2 * SB
ACC_ROWS = 51200
ZSTRIPE = ACC_ROWS // NS

_CPARAMS = pltpu.CompilerParams(
    use_tc_tiling_on_sc=False, needs_layout_passes=False)


def _partition_kernel(nsbp):
  ch = nsbp * SB
  rcap = ch + SB
  mesh = plsc.VectorSubcoreMesh(core_axis_name="c", subcore_axis_name="s")

  @functools.partial(
      pl.kernel,
      mesh=mesh,
      out_type=(
          jax.ShapeDtypeStruct((2, NW, rcap), jnp.int32),
          jax.ShapeDtypeStruct((2, NW, rcap), jnp.int32),
          jax.ShapeDtypeStruct((2, NW, rcap), jnp.float32),
          jax.ShapeDtypeStruct((2, NW, 16), jnp.int32),
      ),
      compiler_params=_CPARAMS,
      scratch_types=[
          pltpu.VMEM((2, SB), jnp.int32),
          pltpu.VMEM((2, SB), jnp.int32),
          pltpu.VMEM((2, SB), jnp.float32),
          pltpu.VMEM((2, RING), jnp.int32),
          pltpu.VMEM((2, RING), jnp.int32),
          pltpu.VMEM((2, RING), jnp.float32),
          pltpu.VMEM((16,), jnp.int32),
          pltpu.SemaphoreType.DMA,
      ],
  )
  def body(srcs, dsts, vals, psrc, pdl, pval, pcnt,
           src_v, dst_v, val_v, rsrc, rdl, rval, cntv, ssem):
    c = lax.axis_index("c")
    s = lax.axis_index("s")
    w = s * NC + c
    lane = lax.iota(jnp.int32, 16)
    zero16i = jnp.zeros((16,), jnp.int32)
    zero16f = jnp.zeros((16,), jnp.float32)
    dummy16 = zero16i + HALF
    rmask = jnp.int32(RING - 1)

    def stage(b, buf):
      base = w * ch + b * SB
      pltpu.async_copy(srcs.at[pl.ds(base, SB)], src_v.at[buf], ssem)
      pltpu.async_copy(dsts.at[pl.ds(base, SB)], dst_v.at[buf], ssem)
      pltpu.async_copy(vals.at[pl.ds(base, SB)], val_v.at[buf], ssem)

    stage(0, 0)

    def flush(h, fh):
      fh = pl.multiple_of(fh, SB)
      roff = pl.multiple_of(lax.rem(fh >> 10, 2) * SB, SB)
      pltpu.sync_copy(rsrc.at[h].at[pl.ds(roff, SB)],
                      psrc.at[h].at[w].at[pl.ds(fh, SB)])
      pltpu.sync_copy(rdl.at[h].at[pl.ds(roff, SB)],
                      pdl.at[h].at[w].at[pl.ds(fh, SB)])
      pltpu.sync_copy(rval.at[h].at[pl.ds(roff, SB)],
                      pval.at[h].at[w].at[pl.ds(fh, SB)])

    def super_body(b, carry):
      w0, f0, w1, f1 = carry
      buf = lax.rem(b, 2)
      pltpu.make_async_copy(srcs.at[pl.ds(0, SB)], src_v.at[buf], ssem).wait()
      pltpu.make_async_copy(dsts.at[pl.ds(0, SB)], dst_v.at[buf], ssem).wait()
      pltpu.make_async_copy(vals.at[pl.ds(0, SB)], val_v.at[buf], ssem).wait()

      @pl.when(b < nsbp - 1)
      def _():
        stage(b + 1, 1 - buf)

      def compact(g, cc):
        cw0, cw1 = cc
        sl = pl.ds(g * 16, 16)
        sv = src_v[buf, sl]
        dv = dst_v[buf, sl]
        vv = val_v[buf, sl]
        ok0 = dv < HALF
        ok1 = dv >= HALF
        xi0 = jnp.where(ok0, 1, 0)
        cum0 = plsc.cumsum(xi0)
        idx0 = ((cw0 + cum0) - xi0) & rmask
        plsc.store_scatter(rsrc.at[0], [idx0], sv, mask=ok0)
        plsc.store_scatter(rdl.at[0], [idx0], dv, mask=ok0)
        plsc.store_scatter(rval.at[0], [idx0], vv, mask=ok0)
        xi1 = jnp.where(ok1, 1, 0)
        cum1 = plsc.cumsum(xi1)
        idx1 = ((cw1 + cum1) - xi1) & rmask
        plsc.store_scatter(rsrc.at[1], [idx1], sv, mask=ok1)
        plsc.store_scatter(rdl.at[1], [idx1], dv - HALF, mask=ok1)
        plsc.store_scatter(rval.at[1], [idx1], vv, mask=ok1)
        return (cw0 + cum0[15], cw1 + cum1[15])

      w0, w1 = lax.fori_loop(0, SB // 16, compact, (w0, w1))

      c0 = (w0 - f0) >= SB

      @pl.when(c0)
      def _():
        flush(0, f0)

      f0 = jnp.where(c0, f0 + SB, f0)
      c1 = (w1 - f1) >= SB

      @pl.when(c1)
      def _():
        flush(1, f1)

      f1 = jnp.where(c1, f1 + SB, f1)
      return (w0, f0, w1, f1)

    z = jnp.int32(0)
    w0, f0, w1, f1 = lax.fori_loop(0, nsbp, super_body, (z, z, z, z))

    for h, (wh, fh) in ((0, (w0, f0)), (1, (w1, f1))):
      for g in range(SB // 16):
        idx = (wh + lane + g * 16) & rmask
        plsc.store_scatter(rsrc.at[h], [idx], zero16i)
        plsc.store_scatter(rdl.at[h], [idx], dummy16)
        plsc.store_scatter(rval.at[h], [idx], zero16f)
      whr = ((wh + (SB - 1)) >> 10) << 10
      cond = (whr - fh) >= SB

      @pl.when(cond)
      def _(h=h, fh=fh):
        flush(h, fh)

      cntv[pl.ds(0, 16)] = zero16i + whr
      pltpu.sync_copy(cntv, pcnt.at[h].at[w])

  return body


def _spmm_kernel(nsbp):
  rcap = nsbp * SB + SB
  mesh = plsc.VectorSubcoreMesh(core_axis_name="c", subcore_axis_name="s")

  @functools.partial(
      pl.kernel,
      mesh=mesh,
      out_type=jax.ShapeDtypeStruct((NN, D), jnp.float32),
      compiler_params=_CPARAMS,
      scratch_types=[
          pltpu.VMEM((2, 16), jnp.int32),
          pltpu.VMEM((2, SB), jnp.int32),
          pltpu.VMEM((2, SB), jnp.int32),
          pltpu.VMEM((2, SB), jnp.float32),
          pltpu.VMEM((2, GB), jnp.int32),
          pltpu.VMEM((2 * GB, D), jnp.float32),
          pltpu.VMEM_SHARED((ACC_ROWS, D), jnp.float32),
          pltpu.SemaphoreType.DMA,
          pltpu.SemaphoreType.DMA,
          pltpu.SemaphoreType.DMA,
          pltpu.SemaphoreType.DMA,
          pltpu.SemaphoreType.DMA,
      ],
  )
  def body(emb, psrc, pdl, pval, pcnt, out, cbuf, src_v, dl_v, val_v,
           dloc_v, rows_v, acc, ssem, g0, g1, c0, c1):
    gsems = (g0, g1)
    csems = (c0, c1)
    c = lax.axis_index("c")
    s = lax.axis_index("s")
    zero16 = jnp.zeros((16,), jnp.float32)

    def zrow(i, _):
      rows_v[i, pl.ds(0, 16)] = zero16
      rows_v[i, pl.ds(16, 16)] = zero16
      return 0

    lax.fori_loop(0, GB, zrow, 0)

    def zacc(b, _):
      pltpu.sync_copy(rows_v.at[pl.ds(0, GB)],
                      acc.at[pl.ds(s * ZSTRIPE + b * GB, GB)])
      return 0

    lax.fori_loop(0, ZSTRIPE // GB, zacc, 0)
    plsc.subcore_barrier()

    pltpu.sync_copy(pcnt.at[c].at[pl.ds(2 * s, 2)], cbuf)
    n0 = cbuf[0, pl.ds(0, 16)][0] >> 10
    n1 = cbuf[1, pl.ds(0, 16)][0] >> 10

    def process_block(k, buf):
      sub = k % 2
      boff = sub * GB
      rows_sl = rows_v.at[pl.ds(boff, GB)]
      coff = k * GB
      pltpu.make_async_copy(emb.at[pl.ds(0, GB)], rows_sl, gsems[sub]).wait()

      def scale(q, _):
        eb = coff + q * 16
        vv = val_v[buf, pl.ds(eb, 16)]
        dloc_v[sub, pl.ds(q * 16, 16)] = dl_v[buf, pl.ds(eb, 16)]
        for u in range(16):
          r = boff + q * 16 + u
          rows_v[r, pl.ds(0, 16)] = rows_v[r, pl.ds(0, 16)] * vv[u]
          rows_v[r, pl.ds(16, 16)] = rows_v[r, pl.ds(16, 16)] * vv[u]
        return 0

      lax.fori_loop(0, GB // 16, scale, 0)

      pltpu.async_copy(rows_sl, acc.at[dloc_v.at[sub]], csems[sub], add=True)

      if k >= 1:
        pltpu.make_async_copy(emb.at[pl.ds(0, GB)],
                              acc.at[pl.ds(0, GB)], csems[1 - sub]).wait()

    for ri in range(2):
      reg = 2 * s + ri
      nr = (n0, n1)[ri]

      def stage(b, buf, reg=reg):
        pltpu.async_copy(psrc.at[c].at[reg].at[pl.ds(b * SB, SB)],
                         src_v.at[buf], ssem)
        pltpu.async_copy(pdl.at[c].at[reg].at[pl.ds(b * SB, SB)],
                         dl_v.at[buf], ssem)
        pltpu.async_copy(pval.at[c].at[reg].at[pl.ds(b * SB, SB)],
                         val_v.at[buf], ssem)

      @pl.when(nr > 0)
      def _(stage=stage):
        stage(0, 0)

      def super_body(b, _, stage=stage, nr=nr):
        buf = lax.rem(b, 2)
        pltpu.make_async_copy(psrc.at[0].at[0].at[pl.ds(0, SB)],
                              src_v.at[buf], ssem).wait()
        pltpu.make_async_copy(pdl.at[0].at[0].at[pl.ds(0, SB)],
                              dl_v.at[buf], ssem).wait()
        pltpu.make_async_copy(pval.at[0].at[0].at[pl.ds(0, SB)],
                              val_v.at[buf], ssem).wait()

        @pl.when(b < nr - 1)
        def _():
          stage(b + 1, 1 - buf)

        sv = src_v.at[buf]
        pltpu.async_copy(emb.at[sv.at[pl.ds(0, GB)]],
                         rows_v.at[pl.ds(0, GB)], gsems[0])
        for k in range(NGB):
          process_block(k, buf)
          if k + 1 < NGB:
            pltpu.async_copy(emb.at[sv.at[pl.ds((k + 1) * GB, GB)]],
                             rows_v.at[pl.ds(((k + 1) % 2) * GB, GB)],
                             gsems[(k + 1) % 2])
        pltpu.make_async_copy(emb.at[pl.ds(0, GB)],
                              acc.at[pl.ds(0, GB)],
                              csems[(NGB - 1) % 2]).wait()
        return 0

      lax.fori_loop(0, nr, super_body, 0)

    plsc.subcore_barrier()
    cbase = c * HALF
    pltpu.sync_copy(acc.at[pl.ds(s * 3128, 3080)],
                    out.at[pl.ds(cbase + s * 3128, 3080)])

    @pl.when(s < NS - 1)
    def _():
      pltpu.sync_copy(acc.at[pl.ds(s * 3128 + 3080, 48)],
                      out.at[pl.ds(cbase + s * 3128 + 3080, 48)])

  return body


def _final_kernel(batch):
  pb = batch // NW
  mesh = plsc.VectorSubcoreMesh(core_axis_name="c", subcore_axis_name="s")

  @functools.partial(
      pl.kernel,
      mesh=mesh,
      out_type=jax.ShapeDtypeStruct((batch,), jnp.float32),
      compiler_params=_CPARAMS,
      scratch_types=[
          pltpu.VMEM((pb,), jnp.int32),
          pltpu.VMEM((pb,), jnp.int32),
          pltpu.VMEM((pb,), jnp.int32),
          pltpu.VMEM((pb, D), jnp.float32),
          pltpu.VMEM((pb, D), jnp.float32),
          pltpu.VMEM((pb, D), jnp.float32),
          pltpu.VMEM((pb,), jnp.float32),
          pltpu.SemaphoreType.DMA,
      ],
  )
  def body(ut, it, e1, e2, e3, users, items, out,
           uidx_v, iidx_v, iidx2_v, au, ai, tmp, gam, sem):
    c = lax.axis_index("c")
    s = lax.axis_index("s")
    wid = s * NC + c
    base = wid * pb
    pltpu.sync_copy(users.at[pl.ds(base, pb)], uidx_v)
    pltpu.sync_copy(items.at[pl.ds(base, pb)], iidx_v)
    for i in range(pb // 16):
      iidx2_v[pl.ds(i * 16, 16)] = iidx_v[pl.ds(i * 16, 16)] + NU

    pltpu.async_copy(ut.at[uidx_v], au, sem).wait()
    pltpu.async_copy(it.at[iidx_v], ai, sem).wait()

    for tab in (e1, e2, e3):
      pltpu.async_copy(tab.at[uidx_v], tmp, sem).wait()

      def addu(r, _):
        au[r, pl.ds(0, 16)] = au[r, pl.ds(0, 16)] + tmp[r, pl.ds(0, 16)]
        au[r, pl.ds(16, 16)] = au[r, pl.ds(16, 16)] + tmp[r, pl.ds(16, 16)]
        return 0

      lax.fori_loop(0, pb, addu, 0)
      pltpu.async_copy(tab.at[iidx2_v], tmp, sem).wait()

      def addi(r, _):
        ai[r, pl.ds(0, 16)] = ai[r, pl.ds(0, 16)] + tmp[r, pl.ds(0, 16)]
        ai[r, pl.ds(16, 16)] = ai[r, pl.ds(16, 16)] + tmp[r, pl.ds(16, 16)]
        return 0

      lax.fori_loop(0, pb, addi, 0)

    lane = lax.iota(jnp.int32, 16)
    quarter = jnp.float32(0.25)
    one = jnp.float32(1.0)

    def outer(o, _):
      def inner(k, carry):
        numvec, denvec = carry
        r = o * 16 + k
        u0 = au[r, pl.ds(0, 16)] * quarter
        u1 = au[r, pl.ds(16, 16)] * quarter
        s0 = one / (one + jnp.exp(-u0))
        s1 = one / (one + jnp.exp(-u1))
        i0 = ai[r, pl.ds(0, 16)] * quarter
        i1 = ai[r, pl.ds(16, 16)] * quarter
        x0 = jnp.exp(i0)
        x1 = jnp.exp(i1)
        den = jnp.sum(x0) + jnp.sum(x1)
        num = jnp.sum(s0 * x0) + jnp.sum(s1 * x1)
        hit = lane == k
        return (jnp.where(hit, num, numvec), jnp.where(hit, den, denvec))

      z16 = jnp.zeros((16,), jnp.float32)
      numvec, denvec = lax.fori_loop(0, 16, inner, (z16, z16 + one))
      gam[pl.ds(o * 16, 16)] = numvec / denvec
      return 0

    lax.fori_loop(0, pb // 16, outer, 0)
    pltpu.sync_copy(gam, out.at[pl.ds(base, pb)])

  return body


def kernel(users, items, user_table, item_table, edge_index, edge_vals):
  all0 = jnp.concatenate([user_table, item_table], axis=0)
  ne = edge_vals.shape[0]
  nsb = -(-ne // (NS * SB))
  nsb += nsb & 1
  pad = nsb * NS * SB - ne
  src = jnp.concatenate([edge_index[0], jnp.zeros((pad,), jnp.int32)])
  dst = jnp.concatenate([edge_index[1], jnp.zeros((pad,), jnp.int32)])
  val = jnp.concatenate([edge_vals, jnp.zeros((pad,), jnp.float32)])

  nsbp = nsb // 2
  psrc, pdl, pval, pcnt = _partition_kernel(nsbp)(src, dst, val)
  spmm = _spmm_kernel(nsbp)
  e1 = spmm(all0, psrc, pdl, pval, pcnt)
  e2 = spmm(e1, psrc, pdl, pval, pcnt)
  e3 = spmm(e2, psrc, pdl, pval, pcnt)
  fin = _final_kernel(users.shape[0])
  return fin(user_table, item_table, e1, e2, e3, users, items)


# speedup vs baseline: 2.6767x; 1.0947x over previous
import functools

import jax
import jax.numpy as jnp
from jax import lax
from jax.experimental import pallas as pl
from jax.experimental.pallas import tpu as pltpu
from jax.experimental.pallas import tpu_sc as plsc

NU = 50000
NI = 50000
NN = NU + NI
D = 32
HALF = NN // 2
NC, NS = 2, 16
NW = NC * NS

SB = 1024
GB = 128
NGB = SB // GB
RING = 2 * SB
ACC_ROWS = 51200
ZSTRIPE = ACC_ROWS // NS

_CPARAMS = pltpu.CompilerParams(
    use_tc_tiling_on_sc=False, needs_layout_passes=False)


def _partition_kernel(nsbp):
  ch = nsbp * SB
  rcap = ch + SB
  mesh = plsc.VectorSubcoreMesh(core_axis_name="c", subcore_axis_name="s")

  @functools.partial(
      pl.kernel,
      mesh=mesh,
      out_type=(
          jax.ShapeDtypeStruct((2, NW, rcap), jnp.int32),
          jax.ShapeDtypeStruct((2, NW, rcap), jnp.int32),
          jax.ShapeDtypeStruct((2, NW, rcap), jnp.float32),
          jax.ShapeDtypeStruct((2, NW, 16), jnp.int32),
      ),
      compiler_params=_CPARAMS,
      scratch_types=[
          pltpu.VMEM((2, SB), jnp.int32),
          pltpu.VMEM((2, SB), jnp.int32),
          pltpu.VMEM((2, SB), jnp.float32),
          pltpu.VMEM((2, RING), jnp.int32),
          pltpu.VMEM((2, RING), jnp.int32),
          pltpu.VMEM((2, RING), jnp.float32),
          pltpu.VMEM((16,), jnp.int32),
          pltpu.SemaphoreType.DMA,
      ],
  )
  def body(srcs, dsts, vals, psrc, pdl, pval, pcnt,
           src_v, dst_v, val_v, rsrc, rdl, rval, cntv, ssem):
    c = lax.axis_index("c")
    s = lax.axis_index("s")
    w = s * NC + c
    lane = lax.iota(jnp.int32, 16)
    zero16i = jnp.zeros((16,), jnp.int32)
    zero16f = jnp.zeros((16,), jnp.float32)
    dummy16 = zero16i + HALF
    rmask = jnp.int32(RING - 1)

    def stage(b, buf):
      base = w * ch + b * SB
      pltpu.async_copy(srcs.at[pl.ds(base, SB)], src_v.at[buf], ssem)
      pltpu.async_copy(dsts.at[pl.ds(base, SB)], dst_v.at[buf], ssem)
      pltpu.async_copy(vals.at[pl.ds(base, SB)], val_v.at[buf], ssem)

    stage(0, 0)

    def flush(h, fh):
      fh = pl.multiple_of(fh, SB)
      roff = pl.multiple_of(lax.rem(fh >> 10, 2) * SB, SB)
      pltpu.sync_copy(rsrc.at[h].at[pl.ds(roff, SB)],
                      psrc.at[h].at[w].at[pl.ds(fh, SB)])
      pltpu.sync_copy(rdl.at[h].at[pl.ds(roff, SB)],
                      pdl.at[h].at[w].at[pl.ds(fh, SB)])
      pltpu.sync_copy(rval.at[h].at[pl.ds(roff, SB)],
                      pval.at[h].at[w].at[pl.ds(fh, SB)])

    def super_body(b, carry):
      w0, f0, w1, f1 = carry
      buf = lax.rem(b, 2)
      pltpu.make_async_copy(srcs.at[pl.ds(0, SB)], src_v.at[buf], ssem).wait()
      pltpu.make_async_copy(dsts.at[pl.ds(0, SB)], dst_v.at[buf], ssem).wait()
      pltpu.make_async_copy(vals.at[pl.ds(0, SB)], val_v.at[buf], ssem).wait()

      @pl.when(b < nsbp - 1)
      def _():
        stage(b + 1, 1 - buf)

      def compact(g, cc):
        cw0, cw1 = cc
        sl = pl.ds(g * 16, 16)
        sv = src_v[buf, sl]
        dv = dst_v[buf, sl]
        vv = val_v[buf, sl]
        ok0 = dv < HALF
        ok1 = dv >= HALF
        xi0 = jnp.where(ok0, 1, 0)
        cum0 = plsc.cumsum(xi0)
        idx0 = ((cw0 + cum0) - xi0) & rmask
        plsc.store_scatter(rsrc.at[0], [idx0], sv, mask=ok0)
        plsc.store_scatter(rdl.at[0], [idx0], dv, mask=ok0)
        plsc.store_scatter(rval.at[0], [idx0], vv, mask=ok0)
        xi1 = jnp.where(ok1, 1, 0)
        cum1 = plsc.cumsum(xi1)
        idx1 = ((cw1 + cum1) - xi1) & rmask
        plsc.store_scatter(rsrc.at[1], [idx1], sv, mask=ok1)
        plsc.store_scatter(rdl.at[1], [idx1], dv - HALF, mask=ok1)
        plsc.store_scatter(rval.at[1], [idx1], vv, mask=ok1)
        return (cw0 + cum0[15], cw1 + cum1[15])

      w0, w1 = lax.fori_loop(0, SB // 16, compact, (w0, w1))

      c0 = (w0 - f0) >= SB

      @pl.when(c0)
      def _():
        flush(0, f0)

      f0 = jnp.where(c0, f0 + SB, f0)
      c1 = (w1 - f1) >= SB

      @pl.when(c1)
      def _():
        flush(1, f1)

      f1 = jnp.where(c1, f1 + SB, f1)
      return (w0, f0, w1, f1)

    z = jnp.int32(0)
    w0, f0, w1, f1 = lax.fori_loop(0, nsbp, super_body, (z, z, z, z))

    for h, (wh, fh) in ((0, (w0, f0)), (1, (w1, f1))):
      for g in range(SB // 16):
        idx = (wh + lane + g * 16) & rmask
        plsc.store_scatter(rsrc.at[h], [idx], zero16i)
        plsc.store_scatter(rdl.at[h], [idx], dummy16)
        plsc.store_scatter(rval.at[h], [idx], zero16f)
      whr = ((wh + (SB - 1)) >> 10) << 10
      cond = (whr - fh) >= SB

      @pl.when(cond)
      def _(h=h, fh=fh):
        flush(h, fh)

      cntv[pl.ds(0, 16)] = zero16i + whr
      pltpu.sync_copy(cntv, pcnt.at[h].at[w])

  return body


def _spmm_kernel(nsbp):
  rcap = nsbp * SB + SB
  mesh = plsc.VectorSubcoreMesh(core_axis_name="c", subcore_axis_name="s")

  @functools.partial(
      pl.kernel,
      mesh=mesh,
      out_type=jax.ShapeDtypeStruct((NN, D), jnp.float32),
      compiler_params=_CPARAMS,
      scratch_types=[
          pltpu.VMEM((2, 16), jnp.int32),
          pltpu.VMEM((2, SB), jnp.int32),
          pltpu.VMEM((2, SB), jnp.int32),
          pltpu.VMEM((2, SB), jnp.float32),
          pltpu.VMEM((4, GB), jnp.int32),
          pltpu.VMEM((4 * GB, D), jnp.float32),
          pltpu.VMEM_SHARED((ACC_ROWS, D), jnp.float32),
          pltpu.SemaphoreType.DMA,
          pltpu.SemaphoreType.DMA,
          pltpu.SemaphoreType.DMA,
          pltpu.SemaphoreType.DMA,
          pltpu.SemaphoreType.DMA,
          pltpu.SemaphoreType.DMA,
          pltpu.SemaphoreType.DMA,
          pltpu.SemaphoreType.DMA,
          pltpu.SemaphoreType.DMA,
      ],
  )
  def body(emb, psrc, pdl, pval, pcnt, out, cbuf, src_v, dl_v, val_v,
           dloc_v, rows_v, acc, ssem, g0, g1, g2, g3, c0, c1, c2, c3):
    gsems = (g0, g1, g2, g3)
    csems = (c0, c1, c2, c3)
    c = lax.axis_index("c")
    s = lax.axis_index("s")
    zero16 = jnp.zeros((16,), jnp.float32)

    def zrow(i, _):
      rows_v[i, pl.ds(0, 16)] = zero16
      rows_v[i, pl.ds(16, 16)] = zero16
      return 0

    lax.fori_loop(0, GB, zrow, 0)

    def zacc(b, _):
      pltpu.sync_copy(rows_v.at[pl.ds(0, GB)],
                      acc.at[pl.ds(s * ZSTRIPE + b * GB, GB)])
      return 0

    lax.fori_loop(0, ZSTRIPE // GB, zacc, 0)
    plsc.subcore_barrier()

    pltpu.sync_copy(pcnt.at[c].at[pl.ds(2 * s, 2)], cbuf)
    n0 = cbuf[0, pl.ds(0, 16)][0] >> 10
    n1 = cbuf[1, pl.ds(0, 16)][0] >> 10

    def process_block(k, buf, sv):
      sub = k % 4
      boff = sub * GB
      rows_sl = rows_v.at[pl.ds(boff, GB)]
      coff = k * GB
      pltpu.make_async_copy(emb.at[pl.ds(0, GB)], rows_sl, gsems[sub]).wait()

      if k + 1 < NGB:
        ns = (k + 1) % 4
        if k + 1 >= 4:
          pltpu.make_async_copy(emb.at[pl.ds(0, GB)],
                                acc.at[pl.ds(0, GB)], csems[ns]).wait()
        pltpu.async_copy(emb.at[sv.at[pl.ds((k + 1) * GB, GB)]],
                         rows_v.at[pl.ds(ns * GB, GB)], gsems[ns])

      def scale(q, _):
        eb = coff + q * 16
        vv = val_v[buf, pl.ds(eb, 16)]
        dloc_v[sub, pl.ds(q * 16, 16)] = dl_v[buf, pl.ds(eb, 16)]
        for u in range(16):
          r = boff + q * 16 + u
          rows_v[r, pl.ds(0, 16)] = rows_v[r, pl.ds(0, 16)] * vv[u]
          rows_v[r, pl.ds(16, 16)] = rows_v[r, pl.ds(16, 16)] * vv[u]
        return 0

      lax.fori_loop(0, GB // 16, scale, 0)

      pltpu.async_copy(rows_sl, acc.at[dloc_v.at[sub]], csems[sub], add=True)

    for ri in range(2):
      reg = 2 * s + ri
      nr = (n0, n1)[ri]

      def stage(b, buf, reg=reg):
        pltpu.async_copy(psrc.at[c].at[reg].at[pl.ds(b * SB, SB)],
                         src_v.at[buf], ssem)
        pltpu.async_copy(pdl.at[c].at[reg].at[pl.ds(b * SB, SB)],
                         dl_v.at[buf], ssem)
        pltpu.async_copy(pval.at[c].at[reg].at[pl.ds(b * SB, SB)],
                         val_v.at[buf], ssem)

      @pl.when(nr > 0)
      def _(stage=stage):
        stage(0, 0)

      def super_body(b, _, stage=stage, nr=nr):
        buf = lax.rem(b, 2)
        pltpu.make_async_copy(psrc.at[0].at[0].at[pl.ds(0, SB)],
                              src_v.at[buf], ssem).wait()
        pltpu.make_async_copy(pdl.at[0].at[0].at[pl.ds(0, SB)],
                              dl_v.at[buf], ssem).wait()
        pltpu.make_async_copy(pval.at[0].at[0].at[pl.ds(0, SB)],
                              val_v.at[buf], ssem).wait()

        @pl.when(b < nr - 1)
        def _():
          stage(b + 1, 1 - buf)

        sv = src_v.at[buf]
        pltpu.async_copy(emb.at[sv.at[pl.ds(0, GB)]],
                         rows_v.at[pl.ds(0, GB)], gsems[0])
        for k in range(NGB):
          process_block(k, buf, sv)
        for i in range(4):
          pltpu.make_async_copy(emb.at[pl.ds(0, GB)],
                                acc.at[pl.ds(0, GB)],
                                csems[(NGB - 4 + i) % 4]).wait()
        return 0

      lax.fori_loop(0, nr, super_body, 0)

    plsc.subcore_barrier()
    cbase = c * HALF
    pltpu.sync_copy(acc.at[pl.ds(s * 3128, 3080)],
                    out.at[pl.ds(cbase + s * 3128, 3080)])

    @pl.when(s < NS - 1)
    def _():
      pltpu.sync_copy(acc.at[pl.ds(s * 3128 + 3080, 48)],
                      out.at[pl.ds(cbase + s * 3128 + 3080, 48)])

  return body


def _final_kernel(batch):
  pb = batch // NW
  mesh = plsc.VectorSubcoreMesh(core_axis_name="c", subcore_axis_name="s")

  @functools.partial(
      pl.kernel,
      mesh=mesh,
      out_type=jax.ShapeDtypeStruct((batch,), jnp.float32),
      compiler_params=_CPARAMS,
      scratch_types=[
          pltpu.VMEM((pb,), jnp.int32),
          pltpu.VMEM((pb,), jnp.int32),
          pltpu.VMEM((pb,), jnp.int32),
          pltpu.VMEM((pb, D), jnp.float32),
          pltpu.VMEM((pb, D), jnp.float32),
          pltpu.VMEM((pb, D), jnp.float32),
          pltpu.VMEM((pb,), jnp.float32),
          pltpu.SemaphoreType.DMA,
      ],
  )
  def body(ut, it, e1, e2, e3, users, items, out,
           uidx_v, iidx_v, iidx2_v, au, ai, tmp, gam, sem):
    c = lax.axis_index("c")
    s = lax.axis_index("s")
    wid = s * NC + c
    base = wid * pb
    pltpu.sync_copy(users.at[pl.ds(base, pb)], uidx_v)
    pltpu.sync_copy(items.at[pl.ds(base, pb)], iidx_v)
    for i in range(pb // 16):
      iidx2_v[pl.ds(i * 16, 16)] = iidx_v[pl.ds(i * 16, 16)] + NU

    pltpu.async_copy(ut.at[uidx_v], au, sem).wait()
    pltpu.async_copy(it.at[iidx_v], ai, sem).wait()

    for tab in (e1, e2, e3):
      pltpu.async_copy(tab.at[uidx_v], tmp, sem).wait()

      def addu(r, _):
        au[r, pl.ds(0, 16)] = au[r, pl.ds(0, 16)] + tmp[r, pl.ds(0, 16)]
        au[r, pl.ds(16, 16)] = au[r, pl.ds(16, 16)] + tmp[r, pl.ds(16, 16)]
        return 0

      lax.fori_loop(0, pb, addu, 0)
      pltpu.async_copy(tab.at[iidx2_v], tmp, sem).wait()

      def addi(r, _):
        ai[r, pl.ds(0, 16)] = ai[r, pl.ds(0, 16)] + tmp[r, pl.ds(0, 16)]
        ai[r, pl.ds(16, 16)] = ai[r, pl.ds(16, 16)] + tmp[r, pl.ds(16, 16)]
        return 0

      lax.fori_loop(0, pb, addi, 0)

    lane = lax.iota(jnp.int32, 16)
    quarter = jnp.float32(0.25)
    one = jnp.float32(1.0)

    def outer(o, _):
      def inner(k, carry):
        numvec, denvec = carry
        r = o * 16 + k
        u0 = au[r, pl.ds(0, 16)] * quarter
        u1 = au[r, pl.ds(16, 16)] * quarter
        s0 = one / (one + jnp.exp(-u0))
        s1 = one / (one + jnp.exp(-u1))
        i0 = ai[r, pl.ds(0, 16)] * quarter
        i1 = ai[r, pl.ds(16, 16)] * quarter
        x0 = jnp.exp(i0)
        x1 = jnp.exp(i1)
        den = jnp.sum(x0) + jnp.sum(x1)
        num = jnp.sum(s0 * x0) + jnp.sum(s1 * x1)
        hit = lane == k
        return (jnp.where(hit, num, numvec), jnp.where(hit, den, denvec))

      z16 = jnp.zeros((16,), jnp.float32)
      numvec, denvec = lax.fori_loop(0, 16, inner, (z16, z16 + one))
      gam[pl.ds(o * 16, 16)] = numvec / denvec
      return 0

    lax.fori_loop(0, pb // 16, outer, 0)
    pltpu.sync_copy(gam, out.at[pl.ds(base, pb)])

  return body


def kernel(users, items, user_table, item_table, edge_index, edge_vals):
  all0 = jnp.concatenate([user_table, item_table], axis=0)
  ne = edge_vals.shape[0]
  nsb = -(-ne // (NS * SB))
  nsb += nsb & 1
  pad = nsb * NS * SB - ne
  src = jnp.concatenate([edge_index[0], jnp.zeros((pad,), jnp.int32)])
  dst = jnp.concatenate([edge_index[1], jnp.zeros((pad,), jnp.int32)])
  val = jnp.concatenate([edge_vals, jnp.zeros((pad,), jnp.float32)])

  nsbp = nsb // 2
  psrc, pdl, pval, pcnt = _partition_kernel(nsbp)(src, dst, val)
  spmm = _spmm_kernel(nsbp)
  e1 = spmm(all0, psrc, pdl, pval, pcnt)
  e2 = spmm(e1, psrc, pdl, pval, pcnt)
  e3 = spmm(e2, psrc, pdl, pval, pcnt)
  fin = _final_kernel(users.shape[0])
  return fin(user_table, item_table, e1, e2, e3, users, items)
